# baseline (speedup 1.0000x reference)
import functools

import jax
import jax.numpy as jnp
from jax import lax
from jax.experimental import pallas as pl
from jax.experimental.pallas import tpu as pltpu
from jax.experimental.pallas import tpu_sc as plsc

N = 10000
E = 160000
D = 256
H = 128
NC = 2
NS = 16
NW = NC * NS
NP = 10240
EP = 163840
EPW1 = EP // NW
EPT = EP // NS
CH = 128
NCH = EPT // CH
NPASS = 2
CPP = NCH // NPASS
RPT = NP // NS

_mesh = plsc.VectorSubcoreMesh(
    core_axis_name="c", subcore_axis_name="s", num_cores=NC, num_subcores=NS)

f32 = jnp.float32
i32 = jnp.int32


@functools.partial(
    pl.kernel,
    out_type=(jax.ShapeDtypeStruct((NW * NP,), f32),
              jax.ShapeDtypeStruct((NW * NP,), f32)),
    mesh=_mesh,
    scratch_types=[
        pltpu.VMEM((EPW1,), i32),
        pltpu.VMEM((EPW1,), i32),
        pltpu.VMEM((NP,), f32),
        pltpu.VMEM((NP,), f32),
    ],
    compiler_params=pltpu.CompilerParams(needs_layout_passes=False),
)
def _k1_deg(src_hbm, dst_hbm, hs_hbm, hd_hbm, sv, dv, hs, hd):
    cid = lax.axis_index("c")
    sid = lax.axis_index("s")
    wid = sid * NC + cid
    zero16 = jnp.zeros((16,), f32)

    def zb(i, c):
        hs[pl.ds(i * 16, 16)] = zero16
        hd[pl.ds(i * 16, 16)] = zero16
        return c
    lax.fori_loop(0, NP // 16, zb, 0)

    pltpu.sync_copy(src_hbm.at[pl.ds(wid * EPW1, EPW1)], sv)
    pltpu.sync_copy(dst_hbm.at[pl.ds(wid * EPW1, EPW1)], dv)

    ones = jnp.ones((16,), f32)

    def eb(i, c):
        plsc.addupdate_scatter(hs, [sv[pl.ds(i * 16, 16)]], ones)
        plsc.addupdate_scatter(hd, [dv[pl.ds(i * 16, 16)]], ones)
        return c
    lax.fori_loop(0, EPW1 // 16, eb, 0)

    pltpu.sync_copy(hs, hs_hbm.at[pl.ds(wid * NP, NP)])
    pltpu.sync_copy(hd, hd_hbm.at[pl.ds(wid * NP, NP)])


def _k2_body(hs_ref, x_ref, o_ref):
    deg = jnp.sum(hs_ref[...], axis=0)
    norm = lax.rsqrt(jnp.maximum(deg, 1.0))
    o_ref[:N, :] = x_ref[...] * norm[:N, None]
    o_ref[N:, :] = jnp.zeros((NP - N, H), f32)


def _k2_norm(hs, x):
    return pl.pallas_call(
        _k2_body,
        grid=(2,),
        in_specs=[pl.BlockSpec((NW, NP), lambda c: (0, 0)),
                  pl.BlockSpec((N, H), lambda c: (0, c))],
        out_specs=pl.BlockSpec((NP, H), lambda c: (c, 0)),
        out_shape=jax.ShapeDtypeStruct((2 * NP, H), f32),
    )(hs, x)


@functools.partial(
    pl.kernel,
    out_type=jax.ShapeDtypeStruct((2 * NP, H), f32),
    mesh=_mesh,
    scratch_types=[
        pltpu.VMEM((CPP, CH), i32),
        pltpu.VMEM((CPP, CH), i32),
        pltpu.VMEM((CH, H), f32),
        pltpu.VMEM((CH, H), f32),
        pltpu.VMEM_SHARED((NP, H), f32),
        pltpu.SemaphoreType.DMA,
        pltpu.SemaphoreType.DMA,
    ],
)
def _k3_agg(nf_hbm, src3_hbm, dst3_hbm, z_hbm, raw_hbm,
            sidx, didx, rows_a, rows_b, acc, sem_a, sem_b):
    cid = lax.axis_index("c")
    sid = lax.axis_index("s")
    r0 = sid * RPT

    for k in range(RPT // CH):
        pltpu.sync_copy(z_hbm.at[pl.ds(r0 + k * CH, CH)],
                        acc.at[pl.ds(r0 + k * CH, CH)])
    plsc.subcore_barrier()

    w3 = cid * NS + sid
    for p in range(NPASS):
        pltpu.sync_copy(src3_hbm.at[w3, pl.ds(p * CPP, CPP)], sidx)
        pltpu.sync_copy(dst3_hbm.at[sid, pl.ds(p * CPP, CPP)], didx)

        pltpu.async_copy(nf_hbm.at[sidx.at[0]], rows_a, sem_a)
        pltpu.async_copy(nf_hbm.at[sidx.at[1]], rows_b, sem_b)

        def eb(i, c):
            g = i * 2
            pltpu.make_async_copy(nf_hbm.at[sidx.at[g]], rows_a, sem_a).wait()
            pltpu.sync_copy(rows_a, acc.at[didx.at[g]], add=True)

            @pl.when(g + 2 < CPP)
            def _():
                pltpu.async_copy(nf_hbm.at[sidx.at[g + 2]], rows_a, sem_a)

            pltpu.make_async_copy(nf_hbm.at[sidx.at[g + 1]], rows_b,
                                  sem_b).wait()
            pltpu.sync_copy(rows_b, acc.at[didx.at[g + 1]], add=True)

            @pl.when(g + 3 < CPP)
            def _():
                pltpu.async_copy(nf_hbm.at[sidx.at[g + 3]], rows_b, sem_b)
            return c
        lax.fori_loop(0, CPP // 2, eb, 0)

    plsc.subcore_barrier()
    out0 = cid * NP
    for k in range(RPT // CH):
        pltpu.sync_copy(acc.at[pl.ds(r0 + k * CH, CH)],
                        raw_hbm.at[pl.ds(out0 + r0 + k * CH, CH)])


def _k4_body(hd_ref, raw_ref, o_ref):
    deg = jnp.sum(hd_ref[...], axis=0)
    ni = lax.rsqrt(jnp.maximum(deg, 1.0))
    o_ref[...] = raw_ref[:N, :] * ni[:N, None]


def _k4_final(hd, raw):
    return pl.pallas_call(
        _k4_body,
        grid=(2,),
        in_specs=[pl.BlockSpec((NW, NP), lambda c: (0, 0)),
                  pl.BlockSpec((NP, H), lambda c: (c, 0))],
        out_specs=pl.BlockSpec((N, H), lambda c: (0, c)),
        out_shape=jax.ShapeDtypeStruct((N, D), f32),
    )(hd, raw)


def kernel(x, edge_index):
    src = edge_index[0]
    dst = edge_index[1]
    pad = jnp.full((EP - E,), N, dtype=i32)
    src_p = jnp.concatenate([src, pad])
    dst_p = jnp.concatenate([dst, pad])
    src3 = jnp.concatenate([src_p, src_p + NP]).reshape(NW, NCH, CH)
    dst3 = dst_p.reshape(NS, NCH, CH)
    zrows = jnp.zeros((NP, H), f32)

    hs, hd = _k1_deg(src_p, dst_p)
    hs = hs.reshape(NW, NP)
    hd = hd.reshape(NW, NP)
    nf = _k2_norm(hs, x)
    raw = _k3_agg(nf, src3, dst3, zrows)
    return _k4_final(hd, raw)


# speedup vs baseline: 1.1353x; 1.1353x over previous
import functools

import jax
import jax.numpy as jnp
from jax import lax
from jax.experimental import pallas as pl
from jax.experimental.pallas import tpu as pltpu
from jax.experimental.pallas import tpu_sc as plsc

N = 10000
E = 160000
D = 256
H = 128
NC = 2
NS = 16
NW = NC * NS
NP = 10240
EP = 163840
EPW1 = EP // NW
EPT = EP // NS
CH = 128
NCH = EPT // CH
NPASS = 2
CPP = NCH // NPASS
RPT = NP // NS

_mesh = plsc.VectorSubcoreMesh(
    core_axis_name="c", subcore_axis_name="s", num_cores=NC, num_subcores=NS)

f32 = jnp.float32
i32 = jnp.int32


@functools.partial(
    pl.kernel,
    out_type=(jax.ShapeDtypeStruct((NW * NP,), f32),
              jax.ShapeDtypeStruct((NW * NP,), f32)),
    mesh=_mesh,
    scratch_types=[
        pltpu.VMEM((EPW1,), i32),
        pltpu.VMEM((EPW1,), i32),
        pltpu.VMEM((NP,), f32),
        pltpu.VMEM((NP,), f32),
    ],
    compiler_params=pltpu.CompilerParams(needs_layout_passes=False),
)
def _k1_deg(src_hbm, dst_hbm, hs_hbm, hd_hbm, sv, dv, hs, hd):
    cid = lax.axis_index("c")
    sid = lax.axis_index("s")
    wid = sid * NC + cid
    zero16 = jnp.zeros((16,), f32)

    def zb(i, c):
        hs[pl.ds(i * 16, 16)] = zero16
        hd[pl.ds(i * 16, 16)] = zero16
        return c
    lax.fori_loop(0, NP // 16, zb, 0)

    pltpu.sync_copy(src_hbm.at[pl.ds(wid * EPW1, EPW1)], sv)
    pltpu.sync_copy(dst_hbm.at[pl.ds(wid * EPW1, EPW1)], dv)

    ones = jnp.ones((16,), f32)

    def eb(i, c):
        plsc.addupdate_scatter(hs, [sv[pl.ds(i * 16, 16)]], ones)
        plsc.addupdate_scatter(hd, [dv[pl.ds(i * 16, 16)]], ones)
        return c
    lax.fori_loop(0, EPW1 // 16, eb, 0)

    pltpu.sync_copy(hs, hs_hbm.at[pl.ds(wid * NP, NP)])
    pltpu.sync_copy(hd, hd_hbm.at[pl.ds(wid * NP, NP)])


def _k2_body(hs_ref, x_ref, o_ref):
    deg = jnp.sum(hs_ref[...], axis=0)
    norm = lax.rsqrt(jnp.maximum(deg, 1.0))
    o_ref[...] = x_ref[...] * norm[:, None]


def _k2_norm(hs, x_pad):
    return pl.pallas_call(
        _k2_body,
        grid=(2,),
        in_specs=[pl.BlockSpec((NW, NP), lambda c: (0, 0)),
                  pl.BlockSpec((NP, H), lambda c: (0, c))],
        out_specs=pl.BlockSpec((NP, H), lambda c: (c, 0)),
        out_shape=jax.ShapeDtypeStruct((2 * NP, H), f32),
    )(hs, x_pad)


@functools.partial(
    pl.kernel,
    out_type=jax.ShapeDtypeStruct((2 * NP, H), f32),
    mesh=_mesh,
    scratch_types=[
        pltpu.VMEM((CPP, CH), i32),
        pltpu.VMEM((CPP, CH), i32),
        pltpu.VMEM((CH, H), f32),
        pltpu.VMEM((CH, H), f32),
        pltpu.VMEM_SHARED((NP, H), f32),
        pltpu.SemaphoreType.DMA,
        pltpu.SemaphoreType.DMA,
    ],
)
def _k3_agg(nf_hbm, src3_hbm, dst3_hbm, z_hbm, raw_hbm,
            sidx, didx, rows_a, rows_b, acc, sem_a, sem_b):
    cid = lax.axis_index("c")
    sid = lax.axis_index("s")
    r0 = sid * RPT

    for k in range(RPT // CH):
        pltpu.sync_copy(z_hbm.at[pl.ds(r0 + k * CH, CH)],
                        acc.at[pl.ds(r0 + k * CH, CH)])
    plsc.subcore_barrier()

    w3 = cid * NS + sid
    for p in range(NPASS):
        pltpu.sync_copy(src3_hbm.at[w3, pl.ds(p * CPP, CPP)], sidx)
        pltpu.sync_copy(dst3_hbm.at[sid, pl.ds(p * CPP, CPP)], didx)

        pltpu.async_copy(nf_hbm.at[sidx.at[0]], rows_a, sem_a)
        pltpu.async_copy(nf_hbm.at[sidx.at[1]], rows_b, sem_b)

        def eb(i, c):
            g = i * 2
            pltpu.make_async_copy(nf_hbm.at[sidx.at[g]], rows_a, sem_a).wait()
            pltpu.sync_copy(rows_a, acc.at[didx.at[g]], add=True)

            @pl.when(g + 2 < CPP)
            def _():
                pltpu.async_copy(nf_hbm.at[sidx.at[g + 2]], rows_a, sem_a)

            pltpu.make_async_copy(nf_hbm.at[sidx.at[g + 1]], rows_b,
                                  sem_b).wait()
            pltpu.sync_copy(rows_b, acc.at[didx.at[g + 1]], add=True)

            @pl.when(g + 3 < CPP)
            def _():
                pltpu.async_copy(nf_hbm.at[sidx.at[g + 3]], rows_b, sem_b)
            return c
        lax.fori_loop(0, CPP // 2, eb, 0)

    plsc.subcore_barrier()
    out0 = cid * NP
    for k in range(RPT // CH):
        pltpu.sync_copy(acc.at[pl.ds(r0 + k * CH, CH)],
                        raw_hbm.at[pl.ds(out0 + r0 + k * CH, CH)])


def _k4_body(hd_ref, raw_ref, o_ref):
    deg = jnp.sum(hd_ref[...], axis=0)
    ni = lax.rsqrt(jnp.maximum(deg, 1.0))
    o_ref[...] = raw_ref[:N, :] * ni[:N, None]


def _k4_final(hd, raw):
    return pl.pallas_call(
        _k4_body,
        grid=(2,),
        in_specs=[pl.BlockSpec((NW, NP), lambda c: (0, 0)),
                  pl.BlockSpec((NP, H), lambda c: (c, 0))],
        out_specs=pl.BlockSpec((N, H), lambda c: (0, c)),
        out_shape=jax.ShapeDtypeStruct((N, D), f32),
    )(hd, raw)


def kernel(x, edge_index):
    src = edge_index[0]
    dst = edge_index[1]
    pad = jnp.full((EP - E,), N, dtype=i32)
    src_p = jnp.concatenate([src, pad])
    dst_p = jnp.concatenate([dst, pad])
    src3 = jnp.concatenate([src_p, src_p + NP]).reshape(NW, NCH, CH)
    dst3 = dst_p.reshape(NS, NCH, CH)
    x_pad = jnp.pad(x, ((0, NP - N), (0, 0)))
    zrows = jnp.zeros((NP, H), f32)

    hs, hd = _k1_deg(src_p, dst_p)
    hs = hs.reshape(NW, NP)
    hd = hd.reshape(NW, NP)
    nf = _k2_norm(hs, x_pad)
    raw = _k3_agg(nf, src3, dst3, zrows)
    return _k4_final(hd, raw)
